# baseline (speedup 1.0000x reference)
import functools

import jax
import jax.numpy as jnp
from jax import lax
from jax.experimental import pallas as pl
from jax.experimental.pallas import tpu as pltpu
from jax.experimental.pallas import tpu_sc as plsc

B_ROWS = 4096
SEQ = 200
D = 64
B_TOTAL = B_ROWS * SEQ

NUM_CORES = 2
NUM_SUBCORES = 16
NW = NUM_CORES * NUM_SUBCORES
BLK = 128
N_BR = B_ROWS // BLK
N_BLOCKS = SEQ * N_BR
PER_W = N_BLOCKS // NW
NBUF = 2
TILE_WORDS = 8 * BLK
BLOCK_WORDS = 8 * TILE_WORDS

_mesh = plsc.VectorSubcoreMesh(core_axis_name="c", subcore_axis_name="s")


@functools.partial(
    pl.kernel,
    mesh=_mesh,
    out_type=jax.ShapeDtypeStruct((B_TOTAL * D,), jnp.float32),
    scratch_types=(
        [pltpu.VMEM((BLK,), jnp.int32) for _ in range(NBUF)]
        + [pltpu.VMEM((BLK, D), jnp.float32) for _ in range(NBUF)]
        + [pltpu.VMEM((BLOCK_WORDS,), jnp.float32) for _ in range(NBUF)]
        + [pltpu.SemaphoreType.DMA for _ in range(2 * NBUF)]
    ),
    compiler_params=pltpu.CompilerParams(
        use_tc_tiling_on_sc=False, needs_layout_passes=False
    ),
)
def _embed_gather(ids_hbm, table_hbm, out_hbm, *scratch):
    idxb = scratch[:NBUF]
    staged = scratch[NBUF : 2 * NBUF]
    outb = scratch[2 * NBUF : 3 * NBUF]
    sg = scratch[3 * NBUF : 4 * NBUF]
    sw = scratch[4 * NBUF : 5 * NBUF]

    wid = lax.axis_index("s") * NUM_CORES + lax.axis_index("c")
    base = wid * PER_W

    def tb(j):
        fid = base + j
        t = lax.shift_right_logical(fid, 5)
        br = lax.bitwise_and(fid, N_BR - 1)
        return t, br

    def fetch(j, b):
        t, br = tb(j)
        off = t * B_ROWS + br * BLK
        pltpu.sync_copy(ids_hbm.at[pl.ds(off, BLK)], idxb[b])
        pltpu.async_copy(table_hbm.at[idxb[b]], staged[b], sg[b])

    def wait_gather(b):
        pltpu.make_async_copy(table_hbm.at[idxb[b]], staged[b], sg[b]).wait()

    def out_off(j, dr):
        t, br = tb(j)
        return t * (8 * N_BR * TILE_WORDS) + dr * (N_BR * TILE_WORDS) + br * TILE_WORDS

    def start_write(j, b):
        for dr in range(8):
            pltpu.async_copy(
                outb[b].at[pl.ds(dr * TILE_WORDS, TILE_WORDS)],
                out_hbm.at[pl.ds(out_off(j, dr), TILE_WORDS)],
                sw[b],
            )

    def wait_write(j, b):
        for dr in range(8):
            pltpu.make_async_copy(
                outb[b].at[pl.ds(dr * TILE_WORDS, TILE_WORDS)],
                out_hbm.at[pl.ds(out_off(j, dr), TILE_WORDS)],
                sw[b],
            ).wait()

    iota16 = lax.iota(jnp.int32, 16)

    def transpose_block(b):
        def tbody(d, carry):
            cols = jnp.broadcast_to(d, (16,)).astype(jnp.int32)
            for g in range(8):
                rows = iota16 + (16 * g)
                vals = plsc.load_gather(staged[b], [rows, cols])
                outb[b][pl.ds(d * BLK + 16 * g, 16)] = vals
            return carry

        lax.fori_loop(0, D, tbody, 0)

    for b in range(NBUF):
        fetch(b, b)

    def process(j, b, do_wait_write, do_prefetch):
        wait_gather(b)
        if do_wait_write:
            wait_write(j, b)
        transpose_block(b)
        start_write(j, b)
        if do_prefetch:
            fetch(j + NBUF, b)

    for b in range(NBUF):
        process(b, b, False, True)

    def body(o, carry):
        for b in range(NBUF):
            process(NBUF * o + b, b, True, True)
        return carry

    lax.fori_loop(1, PER_W // NBUF - 1, body, 0)

    for b in range(NBUF):
        process(PER_W - NBUF + b, b, True, False)
    for b in range(NBUF):
        wait_write(PER_W - NBUF + b, b)


def kernel(ids, emb_weight):
    ids_t = jnp.transpose(ids).reshape(-1).astype(jnp.int32)
    out1 = _embed_gather(ids_t, emb_weight)
    out5 = out1.reshape(SEQ, 8, N_BR, 8, BLK)
    return out5.transpose(2, 4, 0, 1, 3).reshape(B_ROWS, SEQ, D)


# speedup vs baseline: 1.7382x; 1.7382x over previous
import functools

import jax
import jax.numpy as jnp
from jax import lax
from jax.experimental import pallas as pl
from jax.experimental.pallas import tpu as pltpu
from jax.experimental.pallas import tpu_sc as plsc

B_ROWS = 4096
SEQ = 200
D = 64
PAIR = 2 * D
B_TOTAL = B_ROWS * SEQ

NUM_CORES = 2
NUM_SUBCORES = 16
NW = NUM_CORES * NUM_SUBCORES
PER_W = B_TOTAL // NW
CHUNK = 128
N_CHUNKS = PER_W // CHUNK
NBUF = 4
OUTER = N_CHUNKS // NBUF

_mesh = plsc.VectorSubcoreMesh(core_axis_name="c", subcore_axis_name="s")


@functools.partial(
    pl.kernel,
    mesh=_mesh,
    out_type=jax.ShapeDtypeStruct((B_TOTAL, PAIR), jnp.float32),
    scratch_types=(
        [pltpu.VMEM((PER_W,), jnp.int32)]
        + [pltpu.VMEM((CHUNK, PAIR), jnp.float32) for _ in range(NBUF)]
        + [pltpu.SemaphoreType.DMA for _ in range(2 * NBUF)]
    ),
    compiler_params=pltpu.CompilerParams(use_tc_tiling_on_sc=True),
)
def _gather_pairs(ids_hbm, table_hbm, out_hbm, idx_v, *bufs_and_sems):
    rows = bufs_and_sems[:NBUF]
    sg = bufs_and_sems[NBUF : 2 * NBUF]
    sw = bufs_and_sems[2 * NBUF : 3 * NBUF]

    wid = lax.axis_index("s") * NUM_CORES + lax.axis_index("c")
    base = wid * PER_W

    pltpu.sync_copy(ids_hbm.at[pl.ds(base, PER_W)], idx_v)

    def start_gather(j, b):
        pltpu.async_copy(
            table_hbm.at[idx_v.at[pl.ds(j * CHUNK, CHUNK)]], rows[b], sg[b]
        )

    def wait_gather(j, b):
        pltpu.make_async_copy(
            table_hbm.at[idx_v.at[pl.ds(j * CHUNK, CHUNK)]], rows[b], sg[b]
        ).wait()

    def start_write(j, b):
        pltpu.async_copy(rows[b], out_hbm.at[pl.ds(base + j * CHUNK, CHUNK)], sw[b])

    def wait_write(j, b):
        pltpu.make_async_copy(
            rows[b], out_hbm.at[pl.ds(base + j * CHUNK, CHUNK)], sw[b]
        ).wait()

    for b in range(NBUF):
        start_gather(b, b)

    def outer(o, carry):
        for b in range(NBUF):
            j = o * NBUF + b
            wait_gather(j, b)
            start_write(j, b)
            wait_write(j, b)
            start_gather(j + NBUF, b)
        return carry

    lax.fori_loop(0, OUTER - 1, outer, 0)

    for b in range(NBUF):
        j = (OUTER - 1) * NBUF + b
        wait_gather(j, b)
        start_write(j, b)
    for b in range(NBUF):
        j = (OUTER - 1) * NBUF + b
        wait_write(j, b)


def _select_transpose_body(pairs_ref, out_ref):
    pairs = pairs_ref[0]
    pt = jnp.transpose(pairs, (1, 0))
    out_ref[0] = pt[:D, :]


_select_transpose = pl.pallas_call(
    _select_transpose_body,
    grid=(SEQ,),
    in_specs=[
        pl.BlockSpec((1, B_ROWS, PAIR), lambda t: (t, 0, 0)),
    ],
    out_specs=pl.BlockSpec((1, D, B_ROWS), lambda t: (t, 0, 0)),
    out_shape=jax.ShapeDtypeStruct((SEQ, D, B_ROWS), jnp.float32),
)


def kernel(ids, emb_weight):
    ids_t = jnp.transpose(ids).astype(jnp.int32)
    table128 = jnp.pad(emb_weight, ((0, 0), (0, D)))
    rows = _gather_pairs(ids_t.reshape(-1), table128)
    planes = rows.reshape(SEQ, B_ROWS, PAIR)
    out_t = _select_transpose(planes)
    return out_t.transpose(2, 0, 1)


# speedup vs baseline: 2.3348x; 1.3432x over previous
import functools

import jax
import jax.numpy as jnp
from jax import lax
from jax.experimental import pallas as pl
from jax.experimental.pallas import tpu as pltpu
from jax.experimental.pallas import tpu_sc as plsc

B_ROWS = 4096
SEQ = 200
D = 64
PAIR = 2 * D
VOCAB = 1000000
B_TOTAL = B_ROWS * SEQ

NUM_CORES = 2
NUM_SUBCORES = 16
NW = NUM_CORES * NUM_SUBCORES
PER_W = B_TOTAL // NW
CHUNK = 128
N_CHUNKS = PER_W // CHUNK
NBUF = 4
OUTER = N_CHUNKS // NBUF

_mesh = plsc.VectorSubcoreMesh(core_axis_name="c", subcore_axis_name="s")


_PREP_BK = 16384
_PREP_GRID = -(-VOCAB // _PREP_BK)


def _prep_body(tnat_ref, out_ref):
    block = tnat_ref[...]
    out_ref[:, :D] = jnp.transpose(block, (1, 0))
    out_ref[:, D:] = jnp.zeros((_PREP_BK, D), jnp.float32)


_prep_table = pl.pallas_call(
    _prep_body,
    grid=(_PREP_GRID,),
    in_specs=[pl.BlockSpec((D, _PREP_BK), lambda i: (0, i))],
    out_specs=pl.BlockSpec((_PREP_BK, PAIR), lambda i: (i, 0)),
    out_shape=jax.ShapeDtypeStruct((VOCAB, PAIR), jnp.float32),
)


@functools.partial(
    pl.kernel,
    mesh=_mesh,
    out_type=jax.ShapeDtypeStruct((B_TOTAL, PAIR), jnp.float32),
    scratch_types=(
        [pltpu.VMEM((PER_W,), jnp.int32)]
        + [pltpu.VMEM((CHUNK, PAIR), jnp.float32) for _ in range(NBUF)]
        + [pltpu.SemaphoreType.DMA for _ in range(2 * NBUF)]
    ),
    compiler_params=pltpu.CompilerParams(use_tc_tiling_on_sc=True),
)
def _gather_rows(ids_hbm, table_hbm, out_hbm, idx_v, *bufs_and_sems):
    rows = bufs_and_sems[:NBUF]
    sg = bufs_and_sems[NBUF : 2 * NBUF]
    sw = bufs_and_sems[2 * NBUF : 3 * NBUF]

    wid = lax.axis_index("s") * NUM_CORES + lax.axis_index("c")
    base = wid * PER_W

    pltpu.sync_copy(ids_hbm.at[pl.ds(base, PER_W)], idx_v)

    def start_gather(j, b):
        pltpu.async_copy(
            table_hbm.at[idx_v.at[pl.ds(j * CHUNK, CHUNK)]], rows[b], sg[b]
        )

    def wait_gather(j, b):
        pltpu.make_async_copy(
            table_hbm.at[idx_v.at[pl.ds(j * CHUNK, CHUNK)]], rows[b], sg[b]
        ).wait()

    def start_write(j, b):
        pltpu.async_copy(rows[b], out_hbm.at[pl.ds(base + j * CHUNK, CHUNK)], sw[b])

    def wait_write(j, b):
        pltpu.make_async_copy(
            rows[b], out_hbm.at[pl.ds(base + j * CHUNK, CHUNK)], sw[b]
        ).wait()

    for b in range(NBUF):
        start_gather(b, b)

    def outer(o, carry):
        for b in range(NBUF):
            j = o * NBUF + b
            wait_gather(j, b)
            start_write(j, b)
            wait_write(j, b)
            start_gather(j + NBUF, b)
        return carry

    lax.fori_loop(0, OUTER - 1, outer, 0)

    for b in range(NBUF):
        j = (OUTER - 1) * NBUF + b
        wait_gather(j, b)
        start_write(j, b)
    for b in range(NBUF):
        j = (OUTER - 1) * NBUF + b
        wait_write(j, b)


def _select_transpose_body(pairs_ref, out_ref):
    pairs = pairs_ref[0]
    pt = jnp.transpose(pairs, (1, 0))
    out_ref[0] = pt[:D, :]


_select_transpose = pl.pallas_call(
    _select_transpose_body,
    grid=(SEQ,),
    in_specs=[
        pl.BlockSpec((1, B_ROWS, PAIR), lambda t: (t, 0, 0)),
    ],
    out_specs=pl.BlockSpec((1, D, B_ROWS), lambda t: (t, 0, 0)),
    out_shape=jax.ShapeDtypeStruct((SEQ, D, B_ROWS), jnp.float32),
)


def kernel(ids, emb_weight):
    ids_t = jnp.transpose(ids).astype(jnp.int32)
    tnat = jnp.transpose(emb_weight)
    table128 = _prep_table(tnat)
    rows = _gather_rows(ids_t.reshape(-1), table128)
    planes = rows.reshape(SEQ, B_ROWS, PAIR)
    out_t = _select_transpose(planes)
    return out_t.transpose(2, 0, 1)


# speedup vs baseline: 2.8321x; 1.2130x over previous
import functools

import jax
import jax.numpy as jnp
from jax import lax
from jax.experimental import pallas as pl
from jax.experimental.pallas import tpu as pltpu
from jax.experimental.pallas import tpu_sc as plsc

B_ROWS = 4096
SEQ = 200
D = 64
PAIR = 2 * D
VOCAB = 1000000
B_TOTAL = B_ROWS * SEQ

NUM_CORES = 2
NUM_SUBCORES = 16
NW = NUM_CORES * NUM_SUBCORES
PER_W = B_TOTAL // NW
CHUNK = 128
N_CHUNKS = PER_W // CHUNK
NBUF = 4
OUTER = N_CHUNKS // NBUF

_mesh = plsc.VectorSubcoreMesh(core_axis_name="c", subcore_axis_name="s")


_PREP_BK = 16384
_PREP_GRID = -(-VOCAB // _PREP_BK)


def _prep_body(tnat_ref, out_ref):
    block = tnat_ref[...]
    out_ref[:, :D] = jnp.transpose(block, (1, 0))
    out_ref[:, D:] = jnp.zeros((_PREP_BK, D), jnp.float32)


_prep_table = pl.pallas_call(
    _prep_body,
    grid=(_PREP_GRID,),
    in_specs=[pl.BlockSpec((D, _PREP_BK), lambda i: (0, i))],
    out_specs=pl.BlockSpec((_PREP_BK, PAIR), lambda i: (i, 0)),
    out_shape=jax.ShapeDtypeStruct((VOCAB, PAIR), jnp.float32),
)


@functools.partial(
    pl.kernel,
    mesh=_mesh,
    out_type=jax.ShapeDtypeStruct((B_TOTAL, 2, D), jnp.float32),
    scratch_types=(
        [pltpu.VMEM((PER_W,), jnp.int32)]
        + [pltpu.VMEM((CHUNK, D), jnp.float32) for _ in range(NBUF)]
        + [pltpu.SemaphoreType.DMA for _ in range(2 * NBUF)]
    ),
    compiler_params=pltpu.CompilerParams(use_tc_tiling_on_sc=False),
)
def _gather_rows(ids_hbm, table_hbm, out_hbm, idx_v, *bufs_and_sems):
    rows = bufs_and_sems[:NBUF]
    sg = bufs_and_sems[NBUF : 2 * NBUF]
    sw = bufs_and_sems[2 * NBUF : 3 * NBUF]

    wid = lax.axis_index("s") * NUM_CORES + lax.axis_index("c")
    base = wid * PER_W

    pltpu.sync_copy(ids_hbm.at[pl.ds(base, PER_W)], idx_v)

    def start_gather(j, b):
        pltpu.async_copy(
            table_hbm.at[idx_v.at[pl.ds(j * CHUNK, CHUNK)]], rows[b], sg[b]
        )

    def wait_gather(j, b):
        pltpu.make_async_copy(
            table_hbm.at[idx_v.at[pl.ds(j * CHUNK, CHUNK)]], rows[b], sg[b]
        ).wait()

    def start_write(j, b):
        pltpu.async_copy(
            rows[b], out_hbm.at[pl.ds(base + j * CHUNK, CHUNK), 0], sw[b]
        )

    def wait_write(j, b):
        pltpu.make_async_copy(
            rows[b], out_hbm.at[pl.ds(base + j * CHUNK, CHUNK), 0], sw[b]
        ).wait()

    for b in range(NBUF):
        start_gather(b, b)

    def outer(o, carry):
        for b in range(NBUF):
            j = o * NBUF + b
            wait_gather(j, b)
            start_write(j, b)
            wait_write(j, b)
            start_gather(j + NBUF, b)
        return carry

    lax.fori_loop(0, OUTER - 1, outer, 0)

    for b in range(NBUF):
        j = (OUTER - 1) * NBUF + b
        wait_gather(j, b)
        start_write(j, b)
    for b in range(NBUF):
        j = (OUTER - 1) * NBUF + b
        wait_write(j, b)


def _select_transpose_body(pairs_ref, out_ref):
    pairs = pairs_ref[0]
    pt = jnp.transpose(pairs, (1, 0))
    out_ref[0] = pt[:D, :]


_select_transpose = pl.pallas_call(
    _select_transpose_body,
    grid=(SEQ,),
    in_specs=[
        pl.BlockSpec((1, B_ROWS, PAIR), lambda t: (t, 0, 0)),
    ],
    out_specs=pl.BlockSpec((1, D, B_ROWS), lambda t: (t, 0, 0)),
    out_shape=jax.ShapeDtypeStruct((SEQ, D, B_ROWS), jnp.float32),
)


def kernel(ids, emb_weight):
    ids_t = jnp.transpose(ids).astype(jnp.int32)
    tnat = jnp.transpose(emb_weight)
    table128 = _prep_table(tnat)
    table2 = table128.reshape(2 * VOCAB, D)
    rows = _gather_rows((ids_t * 2).reshape(-1), table2)
    planes = rows.reshape(SEQ, B_ROWS, PAIR)
    out_t = _select_transpose(planes)
    return out_t.transpose(2, 0, 1)
